# initial kernel scaffold (unmeasured)
import jax
import jax.numpy as jnp
from jax import lax
from jax.experimental import pallas as pl
from jax.experimental.pallas import tpu as pltpu

N_DEV = 4
COMM_DTYPE = jnp.bfloat16


def kernel(x, w_mat, scale_x, scale_w):
    m_per, k = x.shape
    n = w_mat.shape[1]
    n_per = n // N_DEV

    def body(x_ref, w_ref, sx_ref, sw_ref, out_ref,
             send_buf, recv_buf, send_sems, recv_sems):
        my = lax.axis_index("i")
        scale = sx_ref[0] * sw_ref[0]

        barrier = pltpu.get_barrier_semaphore()
        for p in range(1, N_DEV):
            pl.semaphore_signal(
                barrier, inc=1,
                device_id=((my + p) % N_DEV,),
                device_id_type=pl.DeviceIdType.MESH,
            )
        pl.semaphore_wait(barrier, N_DEV - 1)

        x_bf = x_ref[...].astype(jnp.bfloat16)

        for b in range(N_DEV):
            w_b = w_ref[:, b * n_per:(b + 1) * n_per].astype(jnp.bfloat16)
            y_b = lax.dot_general(
                x_bf, w_b, (((1,), (0,)), ((), ())),
                preferred_element_type=jnp.float32,
            ) * scale

            @pl.when(b == my)
            def _():
                out_ref[pl.ds(b * m_per, m_per), :] = y_b

            @pl.when(b != my)
            def _():
                send_buf[b] = y_b.astype(COMM_DTYPE)
                slot = (my - b) % N_DEV - 1
                rdma = pltpu.make_async_remote_copy(
                    src_ref=send_buf.at[b],
                    dst_ref=recv_buf.at[slot],
                    send_sem=send_sems.at[b],
                    recv_sem=recv_sems.at[slot],
                    device_id=(b,),
                    device_id_type=pl.DeviceIdType.MESH,
                )
                rdma.start()

        for r in range(1, N_DEV):
            recv = pltpu.make_async_remote_copy(
                src_ref=send_buf.at[0],
                dst_ref=recv_buf.at[r - 1],
                send_sem=send_sems.at[0],
                recv_sem=recv_sems.at[r - 1],
                device_id=(0,),
                device_id_type=pl.DeviceIdType.MESH,
            )
            recv.wait_recv()
            src_dev = (my + r) % N_DEV
            out_ref[pl.ds(src_dev * m_per, m_per), :] = (
                recv_buf[r - 1].astype(jnp.float32))

        for b in range(N_DEV):
            @pl.when(b != my)
            def _():
                send = pltpu.make_async_remote_copy(
                    src_ref=send_buf.at[b],
                    dst_ref=recv_buf.at[0],
                    send_sem=send_sems.at[b],
                    recv_sem=recv_sems.at[0],
                    device_id=(0,),
                    device_id_type=pl.DeviceIdType.MESH,
                )
                send.wait_send()

    return pl.pallas_call(
        body,
        out_shape=jax.ShapeDtypeStruct((N_DEV * m_per, n_per), jnp.float32),
        in_specs=[
            pl.BlockSpec(memory_space=pltpu.VMEM),
            pl.BlockSpec(memory_space=pltpu.VMEM),
            pl.BlockSpec(memory_space=pltpu.SMEM),
            pl.BlockSpec(memory_space=pltpu.SMEM),
        ],
        out_specs=pl.BlockSpec(memory_space=pltpu.VMEM),
        scratch_shapes=[
            pltpu.VMEM((N_DEV, m_per, n_per), COMM_DTYPE),
            pltpu.VMEM((N_DEV - 1, m_per, n_per), COMM_DTYPE),
            pltpu.SemaphoreType.DMA((N_DEV,)),
            pltpu.SemaphoreType.DMA((N_DEV - 1,)),
        ],
        compiler_params=pltpu.CompilerParams(
            collective_id=0,
            vmem_limit_bytes=128 * 1024 * 1024,
        ),
    )(x, w_mat, scale_x, scale_w)


# baseline (device time: 74589 ns/iter reference)
import jax
import jax.numpy as jnp
from jax import lax
from jax.experimental import pallas as pl
from jax.experimental.pallas import tpu as pltpu

N_DEV = 4
COMM_DTYPE = jnp.bfloat16


def kernel(x, w_mat, scale_x, scale_w):
    m_per, k = x.shape
    n = w_mat.shape[1]
    n_per = n // N_DEV

    def body(x_ref, w_hbm, sx_ref, sw_ref, out_ref,
             w_blk, send_buf, recv_buf, w_sem, send_sems, recv_sems):
        my = lax.axis_index("i")
        scale = sx_ref[0] * sw_ref[0]

        barrier = pltpu.get_barrier_semaphore()
        for p in range(1, N_DEV):
            pl.semaphore_signal(
                barrier, inc=1,
                device_id=((my + p) % N_DEV,),
                device_id_type=pl.DeviceIdType.MESH,
            )
        pl.semaphore_wait(barrier, N_DEV - 1)

        x_bf = x_ref[...].astype(jnp.bfloat16)

        for b in range(N_DEV):
            cp = pltpu.make_async_copy(
                w_hbm.at[:, pl.ds(b * n_per, n_per)], w_blk, w_sem)
            cp.start()
            cp.wait()
            y_b = lax.dot_general(
                x_bf, w_blk[...].astype(jnp.bfloat16),
                (((1,), (0,)), ((), ())),
                preferred_element_type=jnp.float32,
            ) * scale

            @pl.when(b == my)
            def _():
                out_ref[pl.ds(b * m_per, m_per), :] = y_b

            @pl.when(b != my)
            def _():
                send_buf[b] = y_b.astype(COMM_DTYPE)
                slot = (my - b) % N_DEV - 1
                rdma = pltpu.make_async_remote_copy(
                    src_ref=send_buf.at[b],
                    dst_ref=recv_buf.at[slot],
                    send_sem=send_sems.at[b],
                    recv_sem=recv_sems.at[slot],
                    device_id=(b,),
                    device_id_type=pl.DeviceIdType.MESH,
                )
                rdma.start()

        for r in range(1, N_DEV):
            recv = pltpu.make_async_remote_copy(
                src_ref=send_buf.at[0],
                dst_ref=recv_buf.at[r - 1],
                send_sem=send_sems.at[0],
                recv_sem=recv_sems.at[r - 1],
                device_id=(0,),
                device_id_type=pl.DeviceIdType.MESH,
            )
            recv.wait_recv()
            src_dev = (my + r) % N_DEV
            out_ref[pl.ds(src_dev * m_per, m_per), :] = (
                recv_buf[r - 1].astype(jnp.float32))

        for b in range(N_DEV):
            @pl.when(b != my)
            def _():
                send = pltpu.make_async_remote_copy(
                    src_ref=send_buf.at[b],
                    dst_ref=recv_buf.at[0],
                    send_sem=send_sems.at[b],
                    recv_sem=recv_sems.at[0],
                    device_id=(0,),
                    device_id_type=pl.DeviceIdType.MESH,
                )
                send.wait_send()

    return pl.pallas_call(
        body,
        out_shape=jax.ShapeDtypeStruct((N_DEV * m_per, n_per), jnp.float32),
        in_specs=[
            pl.BlockSpec(memory_space=pltpu.VMEM),
            pl.BlockSpec(memory_space=pl.ANY),
            pl.BlockSpec(memory_space=pltpu.SMEM),
            pl.BlockSpec(memory_space=pltpu.SMEM),
        ],
        out_specs=pl.BlockSpec(memory_space=pltpu.VMEM),
        scratch_shapes=[
            pltpu.VMEM((k, n_per), jnp.float32),
            pltpu.VMEM((N_DEV, m_per, n_per), COMM_DTYPE),
            pltpu.VMEM((N_DEV - 1, m_per, n_per), COMM_DTYPE),
            pltpu.SemaphoreType.DMA,
            pltpu.SemaphoreType.DMA((N_DEV,)),
            pltpu.SemaphoreType.DMA((N_DEV - 1,)),
        ],
        compiler_params=pltpu.CompilerParams(
            collective_id=0,
            vmem_limit_bytes=64 * 1024 * 1024,
        ),
    )(x, w_mat, scale_x, scale_w)


# device time: 56114 ns/iter; 1.3292x vs baseline; 1.3292x over previous
import jax
import jax.numpy as jnp
from jax import lax
from jax.experimental import pallas as pl
from jax.experimental.pallas import tpu as pltpu

N_DEV = 4
NBUF = 3
COMM_DTYPE = jnp.bfloat16


def kernel(x, w_mat, scale_x, scale_w):
    m_per, k = x.shape
    n = w_mat.shape[1]
    n_per = n // N_DEV
    k_half = k // 2

    def body(x_ref, w_hbm, sx_ref, sw_ref, out_ref,
             w_blk, send_buf, recv_buf, w_sems, send_sems, recv_sems):
        my = lax.axis_index("i")
        scale = sx_ref[0] * sw_ref[0]

        barrier = pltpu.get_barrier_semaphore()
        for p in range(1, N_DEV):
            pl.semaphore_signal(
                barrier, inc=1,
                device_id=((my + p) % N_DEV,),
                device_id_type=pl.DeviceIdType.MESH,
            )

        def w_unit(j):
            p, h = divmod(j, 2)
            t = (my + 1 + p) % N_DEV
            return pltpu.make_async_copy(
                w_hbm.at[pl.ds(h * k_half, k_half), pl.ds(t * n_per, n_per)],
                w_blk.at[j % NBUF],
                w_sems.at[j % NBUF],
            )

        for j in range(NBUF):
            w_unit(j).start()

        x_bf = x_ref[...].astype(jnp.bfloat16)

        for p in range(N_DEV):
            j0, j1 = 2 * p, 2 * p + 1
            w_unit(j0).wait()
            y = lax.dot_general(
                x_bf[:, :k_half], w_blk[j0 % NBUF].astype(jnp.bfloat16),
                (((1,), (0,)), ((), ())),
                preferred_element_type=jnp.float32,
            )
            if j0 + NBUF < 2 * N_DEV:
                w_unit(j0 + NBUF).start()
            w_unit(j1).wait()
            y = y + lax.dot_general(
                x_bf[:, k_half:], w_blk[j1 % NBUF].astype(jnp.bfloat16),
                (((1,), (0,)), ((), ())),
                preferred_element_type=jnp.float32,
            )
            if j1 + NBUF < 2 * N_DEV:
                w_unit(j1 + NBUF).start()
            y = y * scale

            if p < N_DEV - 1:
                t = (my + 1 + p) % N_DEV
                send_buf[p] = y.astype(COMM_DTYPE)
                if p == 0:
                    pl.semaphore_wait(barrier, N_DEV - 1)
                rdma = pltpu.make_async_remote_copy(
                    src_ref=send_buf.at[p],
                    dst_ref=recv_buf.at[p],
                    send_sem=send_sems.at[p],
                    recv_sem=recv_sems.at[p],
                    device_id=(t,),
                    device_id_type=pl.DeviceIdType.MESH,
                )
                rdma.start()
            else:
                out_ref[pl.ds(my * m_per, m_per), :] = y

        for r in range(N_DEV - 1):
            recv = pltpu.make_async_remote_copy(
                src_ref=send_buf.at[r],
                dst_ref=recv_buf.at[r],
                send_sem=send_sems.at[r],
                recv_sem=recv_sems.at[r],
                device_id=(0,),
                device_id_type=pl.DeviceIdType.MESH,
            )
            recv.wait_recv()
            src_dev = (my - 1 - r) % N_DEV
            out_ref[pl.ds(src_dev * m_per, m_per), :] = (
                recv_buf[r].astype(jnp.float32))

        for p in range(N_DEV - 1):
            send = pltpu.make_async_remote_copy(
                src_ref=send_buf.at[p],
                dst_ref=recv_buf.at[p],
                send_sem=send_sems.at[p],
                recv_sem=recv_sems.at[p],
                device_id=(0,),
                device_id_type=pl.DeviceIdType.MESH,
            )
            send.wait_send()

    return pl.pallas_call(
        body,
        out_shape=jax.ShapeDtypeStruct((N_DEV * m_per, n_per), jnp.float32),
        in_specs=[
            pl.BlockSpec(memory_space=pltpu.VMEM),
            pl.BlockSpec(memory_space=pl.ANY),
            pl.BlockSpec(memory_space=pltpu.SMEM),
            pl.BlockSpec(memory_space=pltpu.SMEM),
        ],
        out_specs=pl.BlockSpec(memory_space=pltpu.VMEM),
        scratch_shapes=[
            pltpu.VMEM((NBUF, k_half, n_per), jnp.float32),
            pltpu.VMEM((N_DEV - 1, m_per, n_per), COMM_DTYPE),
            pltpu.VMEM((N_DEV - 1, m_per, n_per), COMM_DTYPE),
            pltpu.SemaphoreType.DMA((NBUF,)),
            pltpu.SemaphoreType.DMA((N_DEV - 1,)),
            pltpu.SemaphoreType.DMA((N_DEV - 1,)),
        ],
        compiler_params=pltpu.CompilerParams(
            collective_id=0,
            vmem_limit_bytes=64 * 1024 * 1024,
        ),
    )(x, w_mat, scale_x, scale_w)
